# baseline (speedup 1.0000x reference)
import jax
import jax.numpy as jnp
from jax import lax
from jax.experimental import pallas as pl
from jax.experimental.pallas import tpu as pltpu

INPUT_DIM = 1024
LATENT_DIM = 64
NUM_EMBEDDINGS = 1024
BATCH = 16384

TILE = 1024
NB = BATCH // TILE
K = 4
SUB = TILE // K


def _e2_body(emb_ref, e2_ref):
    e2_ref[...] = jnp.sum(emb_ref[...] ** 2, axis=0, keepdims=True)


def _vq_body(x_hbm, we_ref, be_ref, emb_ref, e2_ref, wd_ref, bd_ref, out_ref,
             xbuf, sems):
    i = pl.program_id(0)

    def copies(slot, step):
        return [
            pltpu.make_async_copy(
                x_hbm.at[pl.ds(step * TILE + k * SUB, SUB), :],
                xbuf.at[slot, pl.ds(k * SUB, SUB), :],
                sems.at[slot, k])
            for k in range(K)
        ]

    @pl.when(i == 0)
    def _():
        for c in copies(0, 0):
            c.start()

    @pl.when(i + 1 < NB)
    def _():
        for c in copies((i + 1) % 2, i + 1):
            c.start()

    for c in copies(i % 2, i):
        c.wait()

    x = xbuf[i % 2]
    z = jnp.dot(x, we_ref[...], preferred_element_type=jnp.float32) + be_ref[...]
    sim = jnp.dot(z, emb_ref[...], preferred_element_type=jnp.float32)
    d = jnp.sum(z * z, axis=1, keepdims=True) + e2_ref[...] - 2.0 * sim
    idx = jnp.argmin(d, axis=1)
    enc = (lax.broadcasted_iota(jnp.int32, (TILE, NUM_EMBEDDINGS), 1)
           == idx[:, None]).astype(jnp.float32)
    q = lax.dot_general(enc, emb_ref[...], (((1,), (1,)), ((), ())),
                        preferred_element_type=jnp.float32)
    out_ref[...] = (jnp.dot(q, wd_ref[...], preferred_element_type=jnp.float32)
                    + bd_ref[...])


@jax.jit
def kernel(x, W_enc, b_enc, W_emb, W_dec, b_dec):
    full = lambda shape: pl.BlockSpec(shape, lambda i: (0,) * len(shape))
    e2 = pl.pallas_call(
        _e2_body,
        in_specs=[pl.BlockSpec((LATENT_DIM, NUM_EMBEDDINGS), lambda: (0, 0))],
        out_specs=pl.BlockSpec((1, NUM_EMBEDDINGS), lambda: (0, 0)),
        out_shape=jax.ShapeDtypeStruct((1, NUM_EMBEDDINGS), jnp.float32),
    )(W_emb)
    out = pl.pallas_call(
        _vq_body,
        grid=(NB,),
        in_specs=[
            pl.BlockSpec(memory_space=pltpu.MemorySpace.HBM),
            full((INPUT_DIM, LATENT_DIM)),
            full((1, LATENT_DIM)),
            full((LATENT_DIM, NUM_EMBEDDINGS)),
            full((1, NUM_EMBEDDINGS)),
            full((LATENT_DIM, INPUT_DIM)),
            full((1, INPUT_DIM)),
        ],
        out_specs=pl.BlockSpec((TILE, INPUT_DIM), lambda i: (i, 0)),
        out_shape=jax.ShapeDtypeStruct((BATCH, INPUT_DIM), jnp.float32),
        scratch_shapes=[
            pltpu.VMEM((2, TILE, INPUT_DIM), jnp.float32),
            pltpu.SemaphoreType.DMA((2, K)),
        ],
    )(x, W_enc, b_enc.reshape(1, -1), W_emb, e2, W_dec, b_dec.reshape(1, -1))
    return out


# speedup vs baseline: 1.0052x; 1.0052x over previous
import jax
import jax.numpy as jnp
from jax import lax
from jax.experimental import pallas as pl
from jax.experimental.pallas import tpu as pltpu

INPUT_DIM = 1024
LATENT_DIM = 64
NUM_EMBEDDINGS = 1024
BATCH = 16384

TILE = 1024
NB = BATCH // TILE
NBUF = 3
K = 2
SUB = TILE // K


def _e2_body(emb_ref, e2_ref):
    e2_ref[...] = jnp.sum(emb_ref[...] ** 2, axis=0, keepdims=True)


def _vq_body(x_hbm, we_ref, be_ref, emb_ref, e2_ref, wd_ref, bd_ref, out_ref,
             xbuf, sems):
    i = pl.program_id(0)

    def copies(slot, step):
        return [
            pltpu.make_async_copy(
                x_hbm.at[pl.ds(step * TILE + k * SUB, SUB), :],
                xbuf.at[slot, pl.ds(k * SUB, SUB), :],
                sems.at[slot, k])
            for k in range(K)
        ]

    def start(slot, step):
        for c in copies(slot, step):
            c.start()

    @pl.when(i == 0)
    def _():
        for s in range(NBUF):
            start(s, s)

    @pl.when((i > 0) & (i + NBUF - 1 < NB))
    def _():
        start((i + NBUF - 1) % NBUF, i + NBUF - 1)

    for c in copies(i % NBUF, i):
        c.wait()

    x = xbuf[i % NBUF]
    z = jnp.dot(x, we_ref[...], preferred_element_type=jnp.float32) + be_ref[...]
    sim = jnp.dot(z, emb_ref[...], preferred_element_type=jnp.float32)
    d = jnp.sum(z * z, axis=1, keepdims=True) + e2_ref[...] - 2.0 * sim
    idx = jnp.argmin(d, axis=1)
    enc = (lax.broadcasted_iota(jnp.int32, (TILE, NUM_EMBEDDINGS), 1)
           == idx[:, None]).astype(jnp.float32)
    q = lax.dot_general(enc, emb_ref[...], (((1,), (1,)), ((), ())),
                        preferred_element_type=jnp.float32)
    out_ref[...] = (jnp.dot(q, wd_ref[...], preferred_element_type=jnp.float32)
                    + bd_ref[...])


@jax.jit
def kernel(x, W_enc, b_enc, W_emb, W_dec, b_dec):
    full = lambda shape: pl.BlockSpec(shape, lambda i: (0,) * len(shape))
    e2 = pl.pallas_call(
        _e2_body,
        in_specs=[pl.BlockSpec((LATENT_DIM, NUM_EMBEDDINGS), lambda: (0, 0))],
        out_specs=pl.BlockSpec((1, NUM_EMBEDDINGS), lambda: (0, 0)),
        out_shape=jax.ShapeDtypeStruct((1, NUM_EMBEDDINGS), jnp.float32),
    )(W_emb)
    out = pl.pallas_call(
        _vq_body,
        grid=(NB,),
        in_specs=[
            pl.BlockSpec(memory_space=pltpu.MemorySpace.HBM),
            full((INPUT_DIM, LATENT_DIM)),
            full((1, LATENT_DIM)),
            full((LATENT_DIM, NUM_EMBEDDINGS)),
            full((1, NUM_EMBEDDINGS)),
            full((LATENT_DIM, INPUT_DIM)),
            full((1, INPUT_DIM)),
        ],
        out_specs=pl.BlockSpec((TILE, INPUT_DIM), lambda i: (i, 0)),
        out_shape=jax.ShapeDtypeStruct((BATCH, INPUT_DIM), jnp.float32),
        scratch_shapes=[
            pltpu.VMEM((NBUF, TILE, INPUT_DIM), jnp.float32),
            pltpu.SemaphoreType.DMA((NBUF, K)),
        ],
    )(x, W_enc, b_enc.reshape(1, -1), W_emb, e2, W_dec, b_dec.reshape(1, -1))
    return out


# speedup vs baseline: 1.0314x; 1.0260x over previous
import jax
import jax.numpy as jnp
from jax import lax
from jax.experimental import pallas as pl

INPUT_DIM = 1024
LATENT_DIM = 64
NUM_EMBEDDINGS = 1024
BATCH = 16384

TILE = 1024


def _vq_body(x_ref, we_ref, be_ref, emb_ref, wd_ref, bd_ref, out_ref):
    x = x_ref[...]
    z = jnp.dot(x, we_ref[...], preferred_element_type=jnp.float32) + be_ref[...]
    sim = jnp.dot(z, emb_ref[...], preferred_element_type=jnp.float32)
    e2 = jnp.sum(emb_ref[...] ** 2, axis=0, keepdims=True)
    d = jnp.sum(z * z, axis=1, keepdims=True) + e2 - 2.0 * sim
    idx = jnp.argmin(d, axis=1)
    enc = (lax.broadcasted_iota(jnp.int32, (TILE, NUM_EMBEDDINGS), 1)
           == idx[:, None]).astype(jnp.float32)
    q = lax.dot_general(enc, emb_ref[...], (((1,), (1,)), ((), ())),
                        preferred_element_type=jnp.float32)
    out_ref[...] = (jnp.dot(q, wd_ref[...], preferred_element_type=jnp.float32)
                    + bd_ref[...])


@jax.jit
def kernel(x, W_enc, b_enc, W_emb, W_dec, b_dec):
    nb = BATCH // TILE
    full = lambda shape: pl.BlockSpec(shape, lambda i: (0,) * len(shape))
    out = pl.pallas_call(
        _vq_body,
        grid=(nb,),
        in_specs=[
            pl.BlockSpec((TILE, INPUT_DIM), lambda i: (i, 0)),
            full((INPUT_DIM, LATENT_DIM)),
            full((1, LATENT_DIM)),
            full((LATENT_DIM, NUM_EMBEDDINGS)),
            full((LATENT_DIM, INPUT_DIM)),
            full((1, INPUT_DIM)),
        ],
        out_specs=pl.BlockSpec((TILE, INPUT_DIM), lambda i: (i, 0)),
        out_shape=jax.ShapeDtypeStruct((BATCH, INPUT_DIM), jnp.float32),
    )(x, W_enc, b_enc.reshape(1, -1), W_emb, W_dec, b_dec.reshape(1, -1))
    return out


# speedup vs baseline: 1.0627x; 1.0303x over previous
import jax
import jax.numpy as jnp
from jax import lax
from jax.experimental import pallas as pl

INPUT_DIM = 1024
LATENT_DIM = 64
NUM_EMBEDDINGS = 1024
BATCH = 16384

TILE = 2048


def _vq_body(x_ref, we_ref, be_ref, emb_ref, wd_ref, bd_ref, out_ref):
    x = x_ref[...]
    z = jnp.dot(x, we_ref[...], preferred_element_type=jnp.float32) + be_ref[...]
    sim = jnp.dot(z, emb_ref[...], preferred_element_type=jnp.float32)
    e2 = jnp.sum(emb_ref[...] ** 2, axis=0, keepdims=True)
    d = jnp.sum(z * z, axis=1, keepdims=True) + e2 - 2.0 * sim
    idx = jnp.argmin(d, axis=1)
    enc = (lax.broadcasted_iota(jnp.int32, (TILE, NUM_EMBEDDINGS), 1)
           == idx[:, None]).astype(jnp.float32)
    q = lax.dot_general(enc, emb_ref[...], (((1,), (1,)), ((), ())),
                        preferred_element_type=jnp.float32)
    out_ref[...] = (jnp.dot(q, wd_ref[...], preferred_element_type=jnp.float32)
                    + bd_ref[...])


@jax.jit
def kernel(x, W_enc, b_enc, W_emb, W_dec, b_dec):
    nb = BATCH // TILE
    full = lambda shape: pl.BlockSpec(shape, lambda i: (0,) * len(shape))
    out = pl.pallas_call(
        _vq_body,
        grid=(nb,),
        in_specs=[
            pl.BlockSpec((TILE, INPUT_DIM), lambda i: (i, 0)),
            full((INPUT_DIM, LATENT_DIM)),
            full((1, LATENT_DIM)),
            full((LATENT_DIM, NUM_EMBEDDINGS)),
            full((LATENT_DIM, INPUT_DIM)),
            full((1, INPUT_DIM)),
        ],
        out_specs=pl.BlockSpec((TILE, INPUT_DIM), lambda i: (i, 0)),
        out_shape=jax.ShapeDtypeStruct((BATCH, INPUT_DIM), jnp.float32),
    )(x, W_enc, b_enc.reshape(1, -1), W_emb, W_dec, b_dec.reshape(1, -1))
    return out


# speedup vs baseline: 1.0636x; 1.0009x over previous
import jax
import jax.numpy as jnp
from jax import lax
from jax.experimental import pallas as pl
from jax.experimental.pallas import tpu as pltpu

INPUT_DIM = 1024
LATENT_DIM = 64
NUM_EMBEDDINGS = 1024
BATCH = 16384

TILE = 2048


def _vq_body(x_ref, we_ref, be_ref, emb_ref, wd_ref, bd_ref, out_ref):
    x = x_ref[...]
    z = jnp.dot(x, we_ref[...], preferred_element_type=jnp.float32) + be_ref[...]
    sim = jnp.dot(z, emb_ref[...], preferred_element_type=jnp.float32)
    e2 = jnp.sum(emb_ref[...] ** 2, axis=0, keepdims=True)
    d = jnp.sum(z * z, axis=1, keepdims=True) + e2 - 2.0 * sim
    idx = jnp.argmin(d, axis=1)
    enc = (lax.broadcasted_iota(jnp.int32, (TILE, NUM_EMBEDDINGS), 1)
           == idx[:, None]).astype(jnp.float32)
    q = lax.dot_general(enc, emb_ref[...], (((1,), (1,)), ((), ())),
                        preferred_element_type=jnp.float32)
    out_ref[...] = (jnp.dot(q, wd_ref[...], preferred_element_type=jnp.float32)
                    + bd_ref[...])


@jax.jit
def kernel(x, W_enc, b_enc, W_emb, W_dec, b_dec):
    nb = BATCH // TILE
    full = lambda shape: pl.BlockSpec(shape, lambda i: (0,) * len(shape))
    out = pl.pallas_call(
        _vq_body,
        grid=(nb,),
        in_specs=[
            pl.BlockSpec((TILE, INPUT_DIM), lambda i: (i, 0)),
            full((INPUT_DIM, LATENT_DIM)),
            full((1, LATENT_DIM)),
            full((LATENT_DIM, NUM_EMBEDDINGS)),
            full((LATENT_DIM, INPUT_DIM)),
            full((1, INPUT_DIM)),
        ],
        out_specs=pl.BlockSpec((TILE, INPUT_DIM), lambda i: (i, 0)),
        out_shape=jax.ShapeDtypeStruct((BATCH, INPUT_DIM), jnp.float32),
        compiler_params=pltpu.CompilerParams(
            dimension_semantics=("parallel",)),
    )(x, W_enc, b_enc.reshape(1, -1), W_emb, W_dec, b_dec.reshape(1, -1))
    return out
